# baseline (speedup 1.0000x reference)
import jax
import jax.numpy as jnp
from jax import lax
from jax.experimental import pallas as pl
from jax.experimental.pallas import tpu as pltpu
from jax.experimental.pallas import tpu_sc as plsc

_T, _E, _D_MODEL, _D_FF = 128, 16, 768, 1536


def _moe_kernel(ids_ref, x_ref, w1_ref, w3_ref, w2_ref, out_ref):
    e = pl.program_id(0)

    mask = ids_ref[...] == e
    xm = jnp.where(mask, x_ref[...], 0.0)

    g = jax.lax.dot_general(xm, w1_ref[0], (((1,), (1,)), ((), ())),
                            preferred_element_type=jnp.float32)
    u = jax.lax.dot_general(xm, w3_ref[0], (((1,), (1,)), ((), ())),
                            preferred_element_type=jnp.float32)
    h = (g * jax.nn.sigmoid(g)) * u
    o = jax.lax.dot_general(h, w2_ref[0], (((1,), (1,)), ((), ())),
                            preferred_element_type=jnp.float32)

    @pl.when(e == 0)
    def _init():
        out_ref[...] = jnp.zeros_like(out_ref)

    out_ref[...] += o


def _tc_moe(ids, x, w1, w3, w2):
    return pl.pallas_call(
        _moe_kernel,
        grid=(_E,),
        in_specs=[
            pl.BlockSpec((_T, 1), lambda e: (0, 0)),
            pl.BlockSpec((_T, _D_MODEL), lambda e: (0, 0)),
            pl.BlockSpec((1, _D_FF, _D_MODEL), lambda e: (e, 0, 0)),
            pl.BlockSpec((1, _D_FF, _D_MODEL), lambda e: (e, 0, 0)),
            pl.BlockSpec((1, _D_MODEL, _D_FF), lambda e: (e, 0, 0)),
        ],
        out_specs=pl.BlockSpec((_T, _D_MODEL), lambda e: (0, 0)),
        out_shape=jax.ShapeDtypeStruct((_T, _D_MODEL), jnp.float32),
        compiler_params=pltpu.CompilerParams(
            dimension_semantics=("arbitrary",),
        ),
    )(ids, x, w1, w3, w2)


def _sc_scale(ids_i32):
    mesh = plsc.VectorSubcoreMesh(core_axis_name="c", subcore_axis_name="s")

    def sc_scale_kernel(ids_hbm, out_hbm, ids_v, out_v):
        wid = lax.axis_index("s") * 2 + lax.axis_index("c")

        @pl.when(wid == 0)
        def _():
            pltpu.sync_copy(ids_hbm, ids_v)
            for i in range(_T // 16):
                v = ids_v[pl.ds(i * 16, 16)]
                out_v[pl.ds(i * 16, 16)] = jnp.where(v < _E, 1.0, 0.5)
            pltpu.sync_copy(out_v, out_hbm)

    f = pl.kernel(
        sc_scale_kernel,
        out_type=jax.ShapeDtypeStruct((_T,), jnp.float32),
        mesh=mesh,
        scratch_types=[
            pltpu.VMEM((_T,), jnp.int32),
            pltpu.VMEM((_T,), jnp.float32),
        ],
    )
    return f(ids_i32)


def kernel(x, token_expert_ids, w1, w3, w2):
    ids_flat = token_expert_ids.astype(jnp.int32)
    ids = ids_flat.reshape(_T, 1)
    tc_out = _tc_moe(ids, x, w1, w3, w2)
    scale = _sc_scale(ids_flat)
    return tc_out * scale.reshape(_T, 1)


# speedup vs baseline: 1.2348x; 1.2348x over previous
import jax
import jax.numpy as jnp
from jax.experimental import pallas as pl
from jax.experimental.pallas import tpu as pltpu

_T, _E, _D_MODEL, _D_FF = 128, 16, 768, 1536


def _moe_kernel(ids_ref, x_ref, w1_ref, w3_ref, w2_ref, out_ref):
    e = pl.program_id(0)

    mask = ids_ref[...] == e
    xm = jnp.where(mask, x_ref[...], 0.0)

    g = jax.lax.dot_general(xm, w1_ref[0], (((1,), (1,)), ((), ())),
                            preferred_element_type=jnp.float32)
    u = jax.lax.dot_general(xm, w3_ref[0], (((1,), (1,)), ((), ())),
                            preferred_element_type=jnp.float32)
    h = (g * jax.nn.sigmoid(g)) * u
    o = jax.lax.dot_general(h, w2_ref[0], (((1,), (1,)), ((), ())),
                            preferred_element_type=jnp.float32)

    @pl.when(e == 0)
    def _init():
        out_ref[...] = jnp.zeros_like(out_ref)

    out_ref[...] += o


def kernel(x, token_expert_ids, w1, w3, w2):
    ids = token_expert_ids.astype(jnp.int32).reshape(_T, 1)
    return pl.pallas_call(
        _moe_kernel,
        grid=(_E,),
        in_specs=[
            pl.BlockSpec((_T, 1), lambda e: (0, 0)),
            pl.BlockSpec((_T, _D_MODEL), lambda e: (0, 0)),
            pl.BlockSpec((1, _D_FF, _D_MODEL), lambda e: (e, 0, 0)),
            pl.BlockSpec((1, _D_FF, _D_MODEL), lambda e: (e, 0, 0)),
            pl.BlockSpec((1, _D_MODEL, _D_FF), lambda e: (e, 0, 0)),
        ],
        out_specs=pl.BlockSpec((_T, _D_MODEL), lambda e: (0, 0)),
        out_shape=jax.ShapeDtypeStruct((_T, _D_MODEL), jnp.float32),
        compiler_params=pltpu.CompilerParams(
            dimension_semantics=("arbitrary",),
        ),
    )(ids, x, w1, w3, w2)


# speedup vs baseline: 1.2776x; 1.0347x over previous
import jax
import jax.numpy as jnp
from jax.experimental import pallas as pl
from jax.experimental.pallas import tpu as pltpu

_T, _E, _D_MODEL, _D_FF = 128, 16, 768, 1536


def _start(hbm, buf, sems, m, e, s):
    pltpu.make_async_copy(hbm.at[e], buf.at[s], sems.at[m, s]).start()


def _wait(hbm, buf, sems, m, e, s):
    pltpu.make_async_copy(hbm.at[e], buf.at[s], sems.at[m, s]).wait()


def _moe_kernel(ids_ref, x_ref, w1_hbm, w3_hbm, w2_hbm, out_ref,
                w1_buf, w3_buf, w2_buf, sems):
    x = x_ref[...]
    ids = ids_ref[...]

    for m, hbm, buf in ((0, w1_hbm, w1_buf), (1, w3_hbm, w3_buf),
                        (2, w2_hbm, w2_buf)):
        _start(hbm, buf, sems, m, 0, 0)

    out_ref[...] = jnp.zeros_like(out_ref)

    for e in range(_E):
        s = e % 2
        if e + 1 < _E:
            ns = (e + 1) % 2
            for m, hbm, buf in ((0, w1_hbm, w1_buf), (1, w3_hbm, w3_buf),
                                (2, w2_hbm, w2_buf)):
                _start(hbm, buf, sems, m, e + 1, ns)

        xm = jnp.where(ids == e, x, 0.0)

        _wait(w1_hbm, w1_buf, sems, 0, e, s)
        g = jax.lax.dot_general(xm, w1_buf[s], (((1,), (1,)), ((), ())),
                                preferred_element_type=jnp.float32)
        _wait(w3_hbm, w3_buf, sems, 1, e, s)
        u = jax.lax.dot_general(xm, w3_buf[s], (((1,), (1,)), ((), ())),
                                preferred_element_type=jnp.float32)
        h = (g * jax.nn.sigmoid(g)) * u
        _wait(w2_hbm, w2_buf, sems, 2, e, s)
        out_ref[...] += jax.lax.dot_general(h, w2_buf[s],
                                            (((1,), (1,)), ((), ())),
                                            preferred_element_type=jnp.float32)


def kernel(x, token_expert_ids, w1, w3, w2):
    ids = token_expert_ids.astype(jnp.int32).reshape(_T, 1)
    return pl.pallas_call(
        _moe_kernel,
        in_specs=[
            pl.BlockSpec(memory_space=pltpu.VMEM),
            pl.BlockSpec(memory_space=pltpu.VMEM),
            pl.BlockSpec(memory_space=pl.ANY),
            pl.BlockSpec(memory_space=pl.ANY),
            pl.BlockSpec(memory_space=pl.ANY),
        ],
        out_specs=pl.BlockSpec(memory_space=pltpu.VMEM),
        out_shape=jax.ShapeDtypeStruct((_T, _D_MODEL), jnp.float32),
        scratch_shapes=[
            pltpu.VMEM((2, _D_FF, _D_MODEL), jnp.float32),
            pltpu.VMEM((2, _D_FF, _D_MODEL), jnp.float32),
            pltpu.VMEM((2, _D_MODEL, _D_FF), jnp.float32),
            pltpu.SemaphoreType.DMA((3, 2)),
        ],
        compiler_params=pltpu.CompilerParams(
            vmem_limit_bytes=100 * 1024 * 1024,
        ),
    )(ids, x, w1, w3, w2)
